# Initial kernel scaffold; baseline (speedup 1.0000x reference)
import jax
import jax.numpy as jnp
from jax.experimental import pallas as pl


def kernel(pos, box, batch, precomputed_edge_index, precomputed_shifts_idx):
    raise NotImplementedError("write your pallas kernel here")


# speedup vs baseline: 25.4494x; 25.4494x over previous
import functools

import jax
import jax.numpy as jnp
from jax import lax
from jax.experimental import pallas as pl
from jax.experimental.pallas import tpu as pltpu
from jax.experimental.pallas import tpu_sc as plsc

NC, NS, L = 2, 16, 16
NW = NC * NS

C = 2000

_MESH = dict(core_axis_name="c", subcore_axis_name="s", num_cores=NC,
             num_subcores=NS)
_CPARAMS = pltpu.CompilerParams(needs_layout_passes=False, use_tc_tiling_on_sc=False)


def _wid():
    return lax.axis_index("s") * NC + lax.axis_index("c")


def _bf16_round(x):
    b = plsc.bitcast(x, jnp.int32)
    rb = (b + 0x7FFF + ((b >> 16) & 1)) & jnp.int32(-65536)
    return plsc.bitcast(rb, jnp.float32)


def _atom_kernel_body(apt, pos_hbm, posb_hbm, batch_hbm, inv_hbm, box_hbm,
                      t_hbm, z_hbm, posw_hbm, a8_hbm, posv, posbv, batchv,
                      invv, boxv, tv, zv, poswv, a8v):
    w = _wid()
    base = w * apt
    pltpu.sync_copy(pos_hbm.at[pl.ds(3 * base, 3 * apt)], posv)
    pltpu.sync_copy(posb_hbm.at[pl.ds(3 * base, 3 * apt)], posbv)
    pltpu.sync_copy(batch_hbm.at[pl.ds(base, apt)], batchv)
    pltpu.sync_copy(inv_hbm, invv)
    pltpu.sync_copy(box_hbm, boxv)
    pltpu.sync_copy(t_hbm, tv)
    pltpu.sync_copy(z_hbm, zv)

    def group(p, carry):
        ix = lax.iota(jnp.int32, 16)
        g = batchv[pl.ds(16 * p, 16)]
        z = plsc.load_gather(zv, [g])
        pc = [plsc.load_gather(posv, [48 * p + 3 * ix + c]) for c in range(3)]
        pb = [plsc.load_gather(posbv, [48 * p + 3 * ix + c]) for c in range(3)]
        g9 = g * 9
        f = [pb[0] * plsc.load_gather(invv, [g9 + j])
             + (pb[1] * plsc.load_gather(invv, [g9 + 3 + j])
                + pb[2] * plsc.load_gather(invv, [g9 + 6 + j]))
             for j in range(3)]
        wf = []
        for j in range(3):
            r = lax.rem(f[j], jnp.float32(1.0))
            wf.append(_bf16_round(jnp.where(r < 0, r + 1.0, r)))
        pw = []
        for c in range(3):
            v = (wf[0] * plsc.load_gather(boxv, [g9 + c])
                 + (wf[1] * plsc.load_gather(boxv, [g9 + 3 + c])
                    + wf[2] * plsc.load_gather(boxv, [g9 + 6 + c])))
            pw.append(jnp.where(z > 0, pc[c], v))
        rows = 16 * p + ix
        col = jnp.zeros((16,), jnp.int32)
        for c in range(3):
            plsc.store_scatter(poswv, [rows, col + c], pw[c])
        g24 = g * 24
        rows8 = rows * 8
        for s in range(8):
            for c in range(3):
                tsc = plsc.load_gather(tv, [g24 + 3 * s + c])
                plsc.store_scatter(a8v, [rows8 + s, col + c], pw[c] - tsc)
        return carry

    lax.fori_loop(0, apt // 16, group, 0)
    pltpu.sync_copy(poswv, posw_hbm.at[pl.ds(base, apt), :])
    pltpu.sync_copy(a8v, a8_hbm.at[pl.ds(8 * base, 8 * apt), :])


def _edge_kernel_body(ept, ei_hbm, sfl_hbm, posw_hbm, a8_hbm, w_hbm, vec_hbm,
                      iv, jv, sv, kv, pjv, aiv, wv, vecv, sem):
    w = _wid()
    nch = ept // C

    def chunk(c, carry):
        base = w * ept + c * C
        pltpu.sync_copy(ei_hbm.at[0, pl.ds(base, C)], iv)
        pltpu.sync_copy(ei_hbm.at[1, pl.ds(base, C)], jv)
        pltpu.sync_copy(sfl_hbm.at[pl.ds(3 * base, 3 * C)], sv)

        def kb(p, cy):
            ix = lax.iota(jnp.int32, 16)
            i16 = iv[pl.ds(16 * p, 16)]
            s0 = plsc.load_gather(sv, [48 * p + 3 * ix])
            s1 = plsc.load_gather(sv, [48 * p + 3 * ix + 1])
            s2 = plsc.load_gather(sv, [48 * p + 3 * ix + 2])
            kv[pl.ds(16 * p, 16)] = i16 * 8 + s0 * 4 + s1 * 2 + s2
            return cy

        lax.fori_loop(0, C // 16, kb, 0)

        cp1 = pltpu.async_copy(posw_hbm.at[jv], pjv, sem)
        cp2 = pltpu.async_copy(a8_hbm.at[kv], aiv, sem)
        cp1.wait()
        cp2.wait()

        def nb(p, cy):
            ix = lax.iota(jnp.int32, 16)
            rows = 16 * p + ix
            col = jnp.zeros((16,), jnp.int32)
            d = []
            for c in range(3):
                xj = plsc.load_gather(pjv, [rows, col + c])
                xa = plsc.load_gather(aiv, [rows, col + c])
                d.append(xj - xa)
            d2 = d[0] * d[0] + d[1] * d[1] + d[2] * d[2]
            ib = plsc.bitcast(d2, jnp.int32)
            y = plsc.bitcast(jnp.int32(0x5F3759DF) - (ib >> 1), jnp.float32)
            for _ in range(3):
                y = y * (1.5 - 0.5 * d2 * y * y)
            wgt = jnp.where(d2 > 0, d2 * y, jnp.float32(0.0))
            wv[pl.ds(16 * p, 16)] = wgt
            for c in range(3):
                plsc.store_scatter(vecv, [rows, col + c], -d[c])
            return cy

        lax.fori_loop(0, C // 16, nb, 0)
        pltpu.sync_copy(wv, w_hbm.at[pl.ds(base, C)])
        pltpu.sync_copy(vecv, vec_hbm.at[pl.ds(base, C), :])
        return carry

    lax.fori_loop(0, nch, chunk, 0)


def kernel(pos, box, batch, precomputed_edge_index, precomputed_shifts_idx):
    N = pos.shape[0]
    G = box.shape[0]
    E = precomputed_edge_index.shape[1]
    assert E % (NW * C) == 0
    ept = E // NW

    ei32 = precomputed_edge_index.astype(jnp.int32)
    sfl = precomputed_shifts_idx.astype(jnp.int32).reshape(-1)
    posf = pos.astype(jnp.float32)
    boxf = box.astype(jnp.float32)

    zflag = jnp.all(boxf == 0, axis=(1, 2)).astype(jnp.float32)
    invb = lax.reduce_precision(jnp.linalg.pinv(boxf), 8, 7)
    boxb = lax.reduce_precision(boxf, 8, 7)
    combos = ((jnp.arange(8)[:, None] >> jnp.array([2, 1, 0])[None, :]) & 1
              ).astype(jnp.float32)
    ttab = (combos[None, :, :, None] * boxb[:, None, :, :]).sum(axis=2)

    apt = -(-N // NW)
    apt = -(-apt // 16) * 16
    ap = NW * apt
    pos_pad = jnp.pad(posf, ((0, ap - N), (0, 0))).reshape(-1)
    posb_pad = lax.reduce_precision(pos_pad, 8, 7)
    batch_pad = jnp.pad(batch.astype(jnp.int32), (0, ap - N))

    atom_k = functools.partial(
        pl.kernel,
        out_type=(
            jax.ShapeDtypeStruct((ap, 8), jnp.float32),
            jax.ShapeDtypeStruct((8 * ap, 8), jnp.float32),
        ),
        mesh=plsc.VectorSubcoreMesh(**_MESH),
        compiler_params=_CPARAMS,
        scratch_types=[
            pltpu.VMEM((3 * apt,), jnp.float32),
            pltpu.VMEM((3 * apt,), jnp.float32),
            pltpu.VMEM((apt,), jnp.int32),
            pltpu.VMEM((9 * G,), jnp.float32),
            pltpu.VMEM((9 * G,), jnp.float32),
            pltpu.VMEM((24 * G,), jnp.float32),
            pltpu.VMEM((G,), jnp.float32),
            pltpu.VMEM((apt, 8), jnp.float32),
            pltpu.VMEM((8 * apt, 8), jnp.float32),
        ],
    )(functools.partial(_atom_kernel_body, apt))
    posw, a8 = atom_k(pos_pad, posb_pad, batch_pad, invb.reshape(-1),
                      boxb.reshape(-1), ttab.reshape(-1), zflag)

    edge_k = functools.partial(
        pl.kernel,
        out_type=(
            jax.ShapeDtypeStruct((E,), jnp.float32),
            jax.ShapeDtypeStruct((E, 3), jnp.float32),
        ),
        mesh=plsc.VectorSubcoreMesh(**_MESH),
        compiler_params=_CPARAMS,
        scratch_types=[
            pltpu.VMEM((C,), jnp.int32),
            pltpu.VMEM((C,), jnp.int32),
            pltpu.VMEM((3 * C,), jnp.int32),
            pltpu.VMEM((C,), jnp.int32),
            pltpu.VMEM((C, 8), jnp.float32),
            pltpu.VMEM((C, 8), jnp.float32),
            pltpu.VMEM((C,), jnp.float32),
            pltpu.VMEM((C, 3), jnp.float32),
            pltpu.SemaphoreType.DMA,
        ],
    )(functools.partial(_edge_kernel_body, ept))
    edge_weight, edge_vec = edge_k(ei32, sfl, posw, a8)

    return (precomputed_edge_index, edge_weight, edge_vec,
            precomputed_shifts_idx)
